# Initial kernel scaffold; baseline (speedup 1.0000x reference)
import jax
import jax.numpy as jnp
from jax.experimental import pallas as pl


def kernel(x, edge_index, batch, Wq1, bq1, Wk1, bk1, Wv1, bv1, Ws1, bs1, Wq2, bq2, Wk2, bk2, Wv2, bv2, Ws2, bs2, Wfc, bfc):
    raise NotImplementedError("write your pallas kernel here")


# speedup vs baseline: 28.7277x; 28.7277x over previous
import functools
import math

import jax
import jax.numpy as jnp
from jax import lax
from jax.experimental import pallas as pl
from jax.experimental.pallas import tpu as pltpu
from jax.experimental.pallas import tpu_sc as plsc

N_NODES = 10000
N_PAD = 10240
N_EDGES = 320000
D_FEAT = 128
HEADS = 4
NUM_GRAPHS = 128
OUT_DIM = 64

_NC = 2
_NS = 16
_ROWS_PER_TILE = N_PAD // _NS
_EDGES_PER_TILE = N_EDGES // (_NC * _NS)
_CHUNK = 80
_NCHUNK = _EDGES_PER_TILE // _CHUNK

_ROW_BLK = 512
_NBLK = N_PAD // _ROW_BLK


def _make_edge_pass(dh):
    c_per_head = dh // HEADS
    inv_sqrt = 1.0 / math.sqrt(c_per_head)
    mesh = plsc.VectorSubcoreMesh(core_axis_name="c", subcore_axis_name="s")

    def body(q_hbm, kv_hbm, src_hbm, dst_hbm,
             num_a, den_a, num_b, den_b,
             dstb, srcb, qrows, kvrows, msgn, msgd, acc_n, acc_d, sem):
        cid = lax.axis_index("c")
        sid = lax.axis_index("s")
        zeros16 = jnp.zeros((16,), jnp.float32)

        for r in range(16):
            for j in range(dh // 16):
                msgn[r, pl.ds(j * 16, 16)] = zeros16
        zbase = sid * _ROWS_PER_TILE

        def zstep(i, carry):
            pltpu.sync_copy(msgn.at[pl.ds(0, 16)],
                            acc_n.at[pl.ds(zbase + i * 16, 16)])
            pltpu.sync_copy(msgn.at[pl.ds(0, 16)],
                            acc_d.at[pl.ds(zbase + i * 16, 16)])
            return carry

        lax.fori_loop(0, _ROWS_PER_TILE // 16, zstep, 0)
        plsc.subcore_barrier()

        wid = sid * _NC + cid
        ebase = wid * _EDGES_PER_TILE
        riota = lax.broadcasted_iota(jnp.int32, (16,), 0)

        def chunk(i, carry):
            off = ebase + i * _CHUNK
            pltpu.sync_copy(dst_hbm.at[pl.ds(off, _CHUNK)], dstb)
            pltpu.sync_copy(src_hbm.at[pl.ds(off, _CHUNK)], srcb)
            pltpu.async_copy(q_hbm.at[dstb], qrows, sem).wait()
            pltpu.async_copy(kv_hbm.at[srcb], kvrows, sem).wait()
            for mb in range(_CHUNK // 16):
                rbase = riota + (mb * 16)
                for h in range(HEADS):
                    acc = jnp.zeros((16,), jnp.float32)
                    for cc in range(c_per_head):
                        j = h * c_per_head + cc
                        cj = jnp.full((16,), j, jnp.int32)
                        qc = plsc.load_gather(qrows, [rbase, cj])
                        kc = plsc.load_gather(kvrows, [rbase, cj])
                        acc = acc + qc * kc
                    w = jnp.exp(acc * inv_sqrt)
                    for cc in range(c_per_head):
                        j = h * c_per_head + cc
                        cj = jnp.full((16,), j, jnp.int32)
                        vj = jnp.full((16,), dh + j, jnp.int32)
                        vc = plsc.load_gather(kvrows, [rbase, vj])
                        plsc.store_scatter(msgn, [rbase, cj], w * vc)
                        plsc.store_scatter(msgd, [rbase, cj], w)
            pltpu.sync_copy(msgn, acc_n.at[dstb], add=True)
            pltpu.sync_copy(msgd, acc_d.at[dstb], add=True)
            return carry

        lax.fori_loop(0, _NCHUNK, chunk, 0)
        plsc.subcore_barrier()

        obase = sid * _ROWS_PER_TILE

        @pl.when(cid == 0)
        def _():
            pltpu.sync_copy(acc_n.at[pl.ds(obase, _ROWS_PER_TILE)],
                            num_a.at[pl.ds(obase, _ROWS_PER_TILE)])
            pltpu.sync_copy(acc_d.at[pl.ds(obase, _ROWS_PER_TILE)],
                            den_a.at[pl.ds(obase, _ROWS_PER_TILE)])

        @pl.when(cid == 1)
        def _():
            pltpu.sync_copy(acc_n.at[pl.ds(obase, _ROWS_PER_TILE)],
                            num_b.at[pl.ds(obase, _ROWS_PER_TILE)])
            pltpu.sync_copy(acc_d.at[pl.ds(obase, _ROWS_PER_TILE)],
                            den_b.at[pl.ds(obase, _ROWS_PER_TILE)])

    node_t = jax.ShapeDtypeStruct((N_PAD, dh), jnp.float32)
    return pl.kernel(
        body,
        out_type=(node_t, node_t, node_t, node_t),
        mesh=mesh,
        compiler_params=pltpu.CompilerParams(
            use_tc_tiling_on_sc=False, needs_layout_passes=False),
        scratch_types=[
            pltpu.VMEM((_CHUNK,), jnp.int32),
            pltpu.VMEM((_CHUNK,), jnp.int32),
            pltpu.VMEM((_CHUNK, dh), jnp.float32),
            pltpu.VMEM((_CHUNK, 2 * dh), jnp.float32),
            pltpu.VMEM((_CHUNK, dh), jnp.float32),
            pltpu.VMEM((_CHUNK, dh), jnp.float32),
            pltpu.VMEM_SHARED((N_PAD, dh), jnp.float32),
            pltpu.VMEM_SHARED((N_PAD, dh), jnp.float32),
            pltpu.SemaphoreType.DMA,
        ],
    )


def _proj_body(x_ref, w_ref, b_ref, o_ref):
    o_ref[...] = (jnp.dot(x_ref[...], w_ref[...],
                          preferred_element_type=jnp.float32) + b_ref[...])


def _mid_body(na_ref, nb_ref, da_ref, db_ref, s_ref, w_ref, b_ref, o_ref):
    num = na_ref[...] + nb_ref[...]
    den = da_ref[...] + db_ref[...]
    h = jnp.maximum(num / (den + 1e-16) + s_ref[...], 0.0)
    o_ref[...] = (jnp.dot(h, w_ref[...],
                          preferred_element_type=jnp.float32) + b_ref[...])


def _tail_body(na_ref, nb_ref, da_ref, db_ref, s_ref, batch_ref, w_ref, b_ref,
               o_ref, sums_ref, cnts_ref):
    i = pl.program_id(0)

    @pl.when(i == 0)
    def _():
        sums_ref[...] = jnp.zeros_like(sums_ref)
        cnts_ref[...] = jnp.zeros_like(cnts_ref)

    num = na_ref[...] + nb_ref[...]
    den = da_ref[...] + db_ref[...]
    h = jnp.maximum(num / (den + 1e-16) + s_ref[...], 0.0)
    bv = batch_ref[0, 0, :].reshape(1, _ROW_BLK)
    gids = lax.broadcasted_iota(jnp.int32, (NUM_GRAPHS, _ROW_BLK), 0)
    onehot_t = (gids == bv).astype(jnp.float32)
    sums_ref[...] += jnp.dot(onehot_t, h, preferred_element_type=jnp.float32)
    cnts_ref[...] += jnp.dot(onehot_t, jnp.ones((_ROW_BLK, 1), jnp.float32),
                             preferred_element_type=jnp.float32)

    @pl.when(i == _NBLK - 1)
    def _():
        pooled = sums_ref[...] / jnp.maximum(cnts_ref[...], 1.0)
        o_ref[...] = (jnp.dot(pooled, w_ref[...],
                              preferred_element_type=jnp.float32) + b_ref[...])


def _proj_call(x_pad, wcat, bcat, d_in, d_out):
    return pl.pallas_call(
        _proj_body,
        grid=(_NBLK,),
        in_specs=[
            pl.BlockSpec((_ROW_BLK, d_in), lambda i: (i, 0)),
            pl.BlockSpec((d_in, d_out), lambda i: (0, 0)),
            pl.BlockSpec((1, d_out), lambda i: (0, 0)),
        ],
        out_specs=pl.BlockSpec((_ROW_BLK, d_out), lambda i: (i, 0)),
        out_shape=jax.ShapeDtypeStruct((N_PAD, d_out), jnp.float32),
    )(x_pad, wcat, bcat)


def _mid_call(na, nb, da, db, s1, wcat, bcat, dh, d_out):
    node = pl.BlockSpec((_ROW_BLK, dh), lambda i: (i, 0))
    return pl.pallas_call(
        _mid_body,
        grid=(_NBLK,),
        in_specs=[
            node, node, node, node, node,
            pl.BlockSpec((dh, d_out), lambda i: (0, 0)),
            pl.BlockSpec((1, d_out), lambda i: (0, 0)),
        ],
        out_specs=pl.BlockSpec((_ROW_BLK, d_out), lambda i: (i, 0)),
        out_shape=jax.ShapeDtypeStruct((N_PAD, d_out), jnp.float32),
    )(na, nb, da, db, s1, wcat, bcat)


def _tail_call(na, nb, da, db, s2, batch3, wfc_t, bfc, dh):
    node = pl.BlockSpec((_ROW_BLK, dh), lambda i: (i, 0))
    return pl.pallas_call(
        _tail_body,
        grid=(_NBLK,),
        in_specs=[
            node, node, node, node, node,
            pl.BlockSpec((1, 1, _ROW_BLK), lambda i: (i, 0, 0)),
            pl.BlockSpec((dh, OUT_DIM), lambda i: (0, 0)),
            pl.BlockSpec((1, OUT_DIM), lambda i: (0, 0)),
        ],
        out_specs=pl.BlockSpec((NUM_GRAPHS, OUT_DIM), lambda i: (0, 0)),
        out_shape=jax.ShapeDtypeStruct((NUM_GRAPHS, OUT_DIM), jnp.float32),
        scratch_shapes=[
            pltpu.VMEM((NUM_GRAPHS, 16), jnp.float32),
            pltpu.VMEM((NUM_GRAPHS, 1), jnp.float32),
        ],
    )(na, nb, da, db, s2, batch3, wfc_t, bfc)


_edge_pass_32 = _make_edge_pass(32)
_edge_pass_16 = _make_edge_pass(16)


def kernel(x, edge_index, batch, Wq1, bq1, Wk1, bk1, Wv1, bv1, Ws1, bs1,
           Wq2, bq2, Wk2, bk2, Wv2, bv2, Ws2, bs2, Wfc, bfc):
    src = edge_index[0].astype(jnp.int32)
    dst = edge_index[1].astype(jnp.int32)
    x_pad = jnp.zeros((N_PAD, D_FEAT), jnp.float32).at[:N_NODES].set(x)

    w1 = jnp.concatenate([Wq1, Wk1, Wv1, Ws1], axis=0).T
    b1 = jnp.concatenate([bq1, bk1, bv1, bs1]).reshape(1, -1)
    out1 = _proj_call(x_pad, w1, b1, D_FEAT, 128)
    q1, kv1, s1 = out1[:, :32], out1[:, 32:96], out1[:, 96:128]

    na, da, nb, db = _edge_pass_32(q1, kv1, src, dst)

    w2 = jnp.concatenate([Wq2, Wk2, Wv2, Ws2], axis=0).T
    b2 = jnp.concatenate([bq2, bk2, bv2, bs2]).reshape(1, -1)
    out2 = _mid_call(na, nb, da, db, s1, w2, b2, 32, 64)
    q2, kv2, s2 = out2[:, :16], out2[:, 16:48], out2[:, 48:64]

    na2, da2, nb2, db2 = _edge_pass_16(q2, kv2, src, dst)

    batch3 = jnp.concatenate(
        [batch.astype(jnp.int32),
         jnp.full((N_PAD - N_NODES,), NUM_GRAPHS, jnp.int32)]
    ).reshape(_NBLK, 1, _ROW_BLK)
    return _tail_call(na2, nb2, da2, db2, s2, batch3,
                      Wfc.T, bfc.reshape(1, -1), 16)


# speedup vs baseline: 37.8027x; 1.3159x over previous
import functools
import math

import jax
import jax.numpy as jnp
from jax import lax
from jax.experimental import pallas as pl
from jax.experimental.pallas import tpu as pltpu
from jax.experimental.pallas import tpu_sc as plsc

N_NODES = 10000
N_PAD = 10240
N_EDGES = 320000
E_PAD = 327680
D_FEAT = 128
HEADS = 4
NUM_GRAPHS = 128
OUT_DIM = 64

_NC = 2
_NS = 16
_ROWS_PER_TILE = N_PAD // _NS
_EDGES_PER_TILE = E_PAD // (_NC * _NS)
_CHUNK = 256
_NCHUNK = _EDGES_PER_TILE // _CHUNK

_ROW_BLK = 512
_NBLK = N_PAD // _ROW_BLK


def _make_edge_pass(dh):
    c_per_head = dh // HEADS
    inv_sqrt = 1.0 / math.sqrt(c_per_head)
    mesh = plsc.VectorSubcoreMesh(core_axis_name="c", subcore_axis_name="s")

    def body(q_hbm, kv_hbm, src_hbm, dst_hbm,
             num_a, den_a, num_b, den_b,
             dstb, srcb, qrows, kvrows, msgn, msgd, acc_n, acc_d,
             sem0, sem1):
        cid = lax.axis_index("c")
        sid = lax.axis_index("s")
        zeros16 = jnp.zeros((16,), jnp.float32)

        for r in range(16):
            for j in range(dh // 16):
                msgn[r, pl.ds(j * 16, 16)] = zeros16
        zbase = sid * _ROWS_PER_TILE

        def zstep(i, carry):
            pltpu.sync_copy(msgn.at[pl.ds(0, 16)],
                            acc_n.at[pl.ds(zbase + i * 16, 16)])
            pltpu.sync_copy(msgn.at[pl.ds(0, 16)],
                            acc_d.at[pl.ds(zbase + i * 16, 16)])
            return carry

        lax.fori_loop(0, _ROWS_PER_TILE // 16, zstep, 0)
        plsc.subcore_barrier()

        wid = sid * _NC + cid
        ebase = wid * _EDGES_PER_TILE
        riota = lax.broadcasted_iota(jnp.int32, (16,), 0)
        sems = (sem0, sem1)

        def issue(i, b):
            off = ebase + i * _CHUNK
            pltpu.sync_copy(dst_hbm.at[pl.ds(off, _CHUNK)], dstb.at[b])
            pltpu.sync_copy(src_hbm.at[pl.ds(off, _CHUNK)], srcb.at[b])
            pltpu.async_copy(q_hbm.at[dstb.at[b]], qrows.at[b], sems[b])
            pltpu.async_copy(kv_hbm.at[srcb.at[b]], kvrows.at[b], sems[b])

        issue(0, 0)

        def compute(i, b):
            pltpu.make_async_copy(q_hbm.at[dstb.at[b]], qrows.at[b],
                                  sems[b]).wait()
            pltpu.make_async_copy(kv_hbm.at[srcb.at[b]], kvrows.at[b],
                                  sems[b]).wait()

            @pl.when(i + 1 < _NCHUNK)
            def _():
                issue(i + 1, 1 - b)

            qr, kvr = qrows.at[b], kvrows.at[b]

            def mbody(mb, carry):
                rbase = riota + mb * 16
                for h in range(HEADS):
                    acc = jnp.zeros((16,), jnp.float32)
                    for cc in range(c_per_head):
                        j = h * c_per_head + cc
                        cj = jnp.full((16,), j, jnp.int32)
                        qc = plsc.load_gather(qr, [rbase, cj])
                        kc = plsc.load_gather(kvr, [rbase, cj])
                        acc = acc + qc * kc
                    w = jnp.exp(acc * inv_sqrt)
                    for cc in range(c_per_head):
                        j = h * c_per_head + cc
                        cj = jnp.full((16,), j, jnp.int32)
                        vj = jnp.full((16,), dh + j, jnp.int32)
                        vc = plsc.load_gather(kvr, [rbase, vj])
                        plsc.store_scatter(msgn, [rbase, cj], w * vc)
                        plsc.store_scatter(msgd, [rbase, cj], w)
                return carry

            lax.fori_loop(0, _CHUNK // 16, mbody, 0)
            pltpu.sync_copy(msgn, acc_n.at[dstb.at[b]], add=True)
            pltpu.sync_copy(msgd, acc_d.at[dstb.at[b]], add=True)

        def outer(o, carry):
            compute(2 * o, 0)

            @pl.when(2 * o + 1 < _NCHUNK)
            def _():
                compute(2 * o + 1, 1)

            return carry

        lax.fori_loop(0, (_NCHUNK + 1) // 2, outer, 0)
        plsc.subcore_barrier()

        obase = sid * _ROWS_PER_TILE

        @pl.when(cid == 0)
        def _():
            pltpu.sync_copy(acc_n.at[pl.ds(obase, _ROWS_PER_TILE)],
                            num_a.at[pl.ds(obase, _ROWS_PER_TILE)])
            pltpu.sync_copy(acc_d.at[pl.ds(obase, _ROWS_PER_TILE)],
                            den_a.at[pl.ds(obase, _ROWS_PER_TILE)])

        @pl.when(cid == 1)
        def _():
            pltpu.sync_copy(acc_n.at[pl.ds(obase, _ROWS_PER_TILE)],
                            num_b.at[pl.ds(obase, _ROWS_PER_TILE)])
            pltpu.sync_copy(acc_d.at[pl.ds(obase, _ROWS_PER_TILE)],
                            den_b.at[pl.ds(obase, _ROWS_PER_TILE)])

    node_t = jax.ShapeDtypeStruct((N_PAD, dh), jnp.float32)
    return pl.kernel(
        body,
        out_type=(node_t, node_t, node_t, node_t),
        mesh=mesh,
        compiler_params=pltpu.CompilerParams(
            use_tc_tiling_on_sc=False, needs_layout_passes=False),
        scratch_types=[
            pltpu.VMEM((2, _CHUNK), jnp.int32),
            pltpu.VMEM((2, _CHUNK), jnp.int32),
            pltpu.VMEM((2, _CHUNK, dh), jnp.float32),
            pltpu.VMEM((2, _CHUNK, 2 * dh), jnp.float32),
            pltpu.VMEM((_CHUNK, dh), jnp.float32),
            pltpu.VMEM((_CHUNK, dh), jnp.float32),
            pltpu.VMEM_SHARED((N_PAD, dh), jnp.float32),
            pltpu.VMEM_SHARED((N_PAD, dh), jnp.float32),
            pltpu.SemaphoreType.DMA,
            pltpu.SemaphoreType.DMA,
        ],
    )


def _proj_body(x_ref, w_ref, b_ref, o_ref):
    o_ref[...] = (jnp.dot(x_ref[...], w_ref[...],
                          preferred_element_type=jnp.float32) + b_ref[...])


def _mid_body(na_ref, nb_ref, da_ref, db_ref, s_ref, w_ref, b_ref, o_ref):
    num = na_ref[...] + nb_ref[...]
    den = da_ref[...] + db_ref[...]
    h = jnp.maximum(num / (den + 1e-16) + s_ref[...], 0.0)
    o_ref[...] = (jnp.dot(h, w_ref[...],
                          preferred_element_type=jnp.float32) + b_ref[...])


def _tail_body(na_ref, nb_ref, da_ref, db_ref, s_ref, batch_ref, w_ref, b_ref,
               o_ref, sums_ref, cnts_ref):
    i = pl.program_id(0)

    @pl.when(i == 0)
    def _():
        sums_ref[...] = jnp.zeros_like(sums_ref)
        cnts_ref[...] = jnp.zeros_like(cnts_ref)

    num = na_ref[...] + nb_ref[...]
    den = da_ref[...] + db_ref[...]
    h = jnp.maximum(num / (den + 1e-16) + s_ref[...], 0.0)
    bv = batch_ref[0, 0, :].reshape(1, _ROW_BLK)
    gids = lax.broadcasted_iota(jnp.int32, (NUM_GRAPHS, _ROW_BLK), 0)
    onehot_t = (gids == bv).astype(jnp.float32)
    sums_ref[...] += jnp.dot(onehot_t, h, preferred_element_type=jnp.float32)
    cnts_ref[...] += jnp.dot(onehot_t, jnp.ones((_ROW_BLK, 1), jnp.float32),
                             preferred_element_type=jnp.float32)

    @pl.when(i == _NBLK - 1)
    def _():
        pooled = sums_ref[...] / jnp.maximum(cnts_ref[...], 1.0)
        o_ref[...] = (jnp.dot(pooled, w_ref[...],
                              preferred_element_type=jnp.float32) + b_ref[...])


def _proj_call(x_pad, wcat, bcat, d_in, d_out):
    return pl.pallas_call(
        _proj_body,
        grid=(_NBLK,),
        in_specs=[
            pl.BlockSpec((_ROW_BLK, d_in), lambda i: (i, 0)),
            pl.BlockSpec((d_in, d_out), lambda i: (0, 0)),
            pl.BlockSpec((1, d_out), lambda i: (0, 0)),
        ],
        out_specs=pl.BlockSpec((_ROW_BLK, d_out), lambda i: (i, 0)),
        out_shape=jax.ShapeDtypeStruct((N_PAD, d_out), jnp.float32),
    )(x_pad, wcat, bcat)


def _mid_call(na, nb, da, db, s1, wcat, bcat, dh, d_out):
    node = pl.BlockSpec((_ROW_BLK, dh), lambda i: (i, 0))
    return pl.pallas_call(
        _mid_body,
        grid=(_NBLK,),
        in_specs=[
            node, node, node, node, node,
            pl.BlockSpec((dh, d_out), lambda i: (0, 0)),
            pl.BlockSpec((1, d_out), lambda i: (0, 0)),
        ],
        out_specs=pl.BlockSpec((_ROW_BLK, d_out), lambda i: (i, 0)),
        out_shape=jax.ShapeDtypeStruct((N_PAD, d_out), jnp.float32),
    )(na, nb, da, db, s1, wcat, bcat)


def _tail_call(na, nb, da, db, s2, batch3, wfc_t, bfc, dh):
    node = pl.BlockSpec((_ROW_BLK, dh), lambda i: (i, 0))
    return pl.pallas_call(
        _tail_body,
        grid=(_NBLK,),
        in_specs=[
            node, node, node, node, node,
            pl.BlockSpec((1, 1, _ROW_BLK), lambda i: (i, 0, 0)),
            pl.BlockSpec((dh, OUT_DIM), lambda i: (0, 0)),
            pl.BlockSpec((1, OUT_DIM), lambda i: (0, 0)),
        ],
        out_specs=pl.BlockSpec((NUM_GRAPHS, OUT_DIM), lambda i: (0, 0)),
        out_shape=jax.ShapeDtypeStruct((NUM_GRAPHS, OUT_DIM), jnp.float32),
        scratch_shapes=[
            pltpu.VMEM((NUM_GRAPHS, 16), jnp.float32),
            pltpu.VMEM((NUM_GRAPHS, 1), jnp.float32),
        ],
    )(na, nb, da, db, s2, batch3, wfc_t, bfc)


_edge_pass_32 = _make_edge_pass(32)
_edge_pass_16 = _make_edge_pass(16)


def kernel(x, edge_index, batch, Wq1, bq1, Wk1, bk1, Wv1, bv1, Ws1, bs1,
           Wq2, bq2, Wk2, bk2, Wv2, bv2, Ws2, bs2, Wfc, bfc):
    src = jnp.concatenate([edge_index[0].astype(jnp.int32),
                           jnp.zeros((E_PAD - N_EDGES,), jnp.int32)])
    dst = jnp.concatenate([edge_index[1].astype(jnp.int32),
                           jnp.full((E_PAD - N_EDGES,), N_NODES, jnp.int32)])
    x_pad = jnp.zeros((N_PAD, D_FEAT), jnp.float32).at[:N_NODES].set(x)

    w1 = jnp.concatenate([Wq1, Wk1, Wv1, Ws1], axis=0).T
    b1 = jnp.concatenate([bq1, bk1, bv1, bs1]).reshape(1, -1)
    out1 = _proj_call(x_pad, w1, b1, D_FEAT, 128)
    q1, kv1, s1 = out1[:, :32], out1[:, 32:96], out1[:, 96:128]

    na, da, nb, db = _edge_pass_32(q1, kv1, src, dst)

    w2 = jnp.concatenate([Wq2, Wk2, Wv2, Ws2], axis=0).T
    b2 = jnp.concatenate([bq2, bk2, bv2, bs2]).reshape(1, -1)
    out2 = _mid_call(na, nb, da, db, s1, w2, b2, 32, 64)
    q2, kv2, s2 = out2[:, :16], out2[:, 16:48], out2[:, 48:64]

    na2, da2, nb2, db2 = _edge_pass_16(q2, kv2, src, dst)

    batch3 = jnp.concatenate(
        [batch.astype(jnp.int32),
         jnp.full((N_PAD - N_NODES,), NUM_GRAPHS, jnp.int32)]
    ).reshape(_NBLK, 1, _ROW_BLK)
    return _tail_call(na2, nb2, da2, db2, s2, batch3,
                      Wfc.T, bfc.reshape(1, -1), 16)


# speedup vs baseline: 45.2345x; 1.1966x over previous
import math

import jax
import jax.numpy as jnp
from jax import lax
from jax.experimental import pallas as pl
from jax.experimental.pallas import tpu as pltpu
from jax.experimental.pallas import tpu_sc as plsc

N_NODES = 10000
N_PAD = 10240
N_EDGES = 320000
E_PAD = 327680
D_FEAT = 128
HEADS = 4
NUM_GRAPHS = 128
OUT_DIM = 64
DEN_W = 16

_NC = 2
_NS = 16
_ROWS_PER_TILE = N_PAD // _NS
_EDGES_PER_TILE = E_PAD // (_NC * _NS)
_CHUNK = 256
_NCHUNK = _EDGES_PER_TILE // _CHUNK

_ROW_BLK = 512
_NBLK = N_PAD // _ROW_BLK


def _make_edge_pass(dh):
    c_per_head = dh // HEADS
    inv_sqrt = 1.0 / math.sqrt(c_per_head)
    mesh = plsc.VectorSubcoreMesh(core_axis_name="c", subcore_axis_name="s")

    def body(q_hbm, kv_hbm, src_hbm, dst_hbm,
             num_a, den_a, num_b, den_b,
             dstb, srcb, qrows, kvrows, msgn, msgd, acc_n, acc_d,
             gsem0, gsem1, tsem0, tsem1):
        cid = lax.axis_index("c")
        sid = lax.axis_index("s")
        zeros16 = jnp.zeros((16,), jnp.float32)
        riota = lax.broadcasted_iota(jnp.int32, (16,), 0)

        def zrow(r, carry):
            rr = jnp.full((16,), r, jnp.int32)
            plsc.store_scatter(msgd.at[0], [rr, riota], zeros16)
            plsc.store_scatter(msgd.at[1], [rr, riota], zeros16)
            return carry

        lax.fori_loop(0, _CHUNK, zrow, 0)
        for r in range(16):
            for j in range(dh // 16):
                msgn[0, r, pl.ds(j * 16, 16)] = zeros16

        zbase = sid * _ROWS_PER_TILE

        def zstep(i, carry):
            pltpu.sync_copy(msgn.at[0].at[pl.ds(0, 16)],
                            acc_n.at[pl.ds(zbase + i * 16, 16)])
            pltpu.sync_copy(msgd.at[0].at[pl.ds(0, 16)],
                            acc_d.at[pl.ds(zbase + i * 16, 16)])
            return carry

        lax.fori_loop(0, _ROWS_PER_TILE // 16, zstep, 0)
        plsc.subcore_barrier()

        wid = sid * _NC + cid
        ebase = wid * _EDGES_PER_TILE
        gsems = (gsem0, gsem1)
        tsems = (tsem0, tsem1)

        def issue(i, b):
            off = ebase + i * _CHUNK
            pltpu.sync_copy(dst_hbm.at[pl.ds(off, _CHUNK)], dstb.at[b])
            pltpu.sync_copy(src_hbm.at[pl.ds(off, _CHUNK)], srcb.at[b])
            pltpu.async_copy(q_hbm.at[dstb.at[b]], qrows.at[b], gsems[b])
            pltpu.async_copy(kv_hbm.at[srcb.at[b]], kvrows.at[b], gsems[b])

        def wait_scatter(b):
            pltpu.make_async_copy(msgn.at[b], acc_n.at[dstb.at[b]],
                                  tsems[b]).wait()
            pltpu.make_async_copy(msgd.at[b], acc_d.at[dstb.at[b]],
                                  tsems[b]).wait()

        issue(0, 0)

        def compute(i, b):
            pltpu.make_async_copy(q_hbm.at[dstb.at[b]], qrows.at[b],
                                  gsems[b]).wait()
            pltpu.make_async_copy(kv_hbm.at[srcb.at[b]], kvrows.at[b],
                                  gsems[b]).wait()

            @pl.when(i + 1 < _NCHUNK)
            def _():
                issue(i + 1, 1 - b)

            qr, kvr = qrows.at[b], kvrows.at[b]
            mn, md = msgn.at[b], msgd.at[b]

            def mbody(mb, carry):
                rbase = riota + mb * 16
                for h in range(HEADS):
                    acc = jnp.zeros((16,), jnp.float32)
                    for cc in range(c_per_head):
                        j = h * c_per_head + cc
                        cj = jnp.full((16,), j, jnp.int32)
                        qc = plsc.load_gather(qr, [rbase, cj])
                        kc = plsc.load_gather(kvr, [rbase, cj])
                        acc = acc + qc * kc
                    w = jnp.exp(acc * inv_sqrt)
                    for cc in range(c_per_head):
                        j = h * c_per_head + cc
                        cj = jnp.full((16,), j, jnp.int32)
                        vj = jnp.full((16,), dh + j, jnp.int32)
                        vc = plsc.load_gather(kvr, [rbase, vj])
                        plsc.store_scatter(mn, [rbase, cj], w * vc)
                    plsc.store_scatter(md, [rbase, jnp.full((16,), h, jnp.int32)], w)
                return carry

            lax.fori_loop(0, _CHUNK // 16, mbody, 0)
            pltpu.sync_copy(msgn.at[b], acc_n.at[dstb.at[b]], add=True)
            pltpu.sync_copy(msgd.at[b], acc_d.at[dstb.at[b]], add=True)

        def outer(o, carry):
            compute(2 * o, 0)
            compute(2 * o + 1, 1)
            return carry

        lax.fori_loop(0, _NCHUNK // 2, outer, 0)
        plsc.subcore_barrier()

        obase = sid * _ROWS_PER_TILE

        @pl.when(cid == 0)
        def _():
            pltpu.sync_copy(acc_n.at[pl.ds(obase, _ROWS_PER_TILE)],
                            num_a.at[pl.ds(obase, _ROWS_PER_TILE)])
            pltpu.sync_copy(acc_d.at[pl.ds(obase, _ROWS_PER_TILE)],
                            den_a.at[pl.ds(obase, _ROWS_PER_TILE)])

        @pl.when(cid == 1)
        def _():
            pltpu.sync_copy(acc_n.at[pl.ds(obase, _ROWS_PER_TILE)],
                            num_b.at[pl.ds(obase, _ROWS_PER_TILE)])
            pltpu.sync_copy(acc_d.at[pl.ds(obase, _ROWS_PER_TILE)],
                            den_b.at[pl.ds(obase, _ROWS_PER_TILE)])

    num_t = jax.ShapeDtypeStruct((N_PAD, dh), jnp.float32)
    den_t = jax.ShapeDtypeStruct((N_PAD, DEN_W), jnp.float32)
    return pl.kernel(
        body,
        out_type=(num_t, den_t, num_t, den_t),
        mesh=mesh,
        compiler_params=pltpu.CompilerParams(
            use_tc_tiling_on_sc=False, needs_layout_passes=False),
        scratch_types=[
            pltpu.VMEM((2, _CHUNK), jnp.int32),
            pltpu.VMEM((2, _CHUNK), jnp.int32),
            pltpu.VMEM((2, _CHUNK, dh), jnp.float32),
            pltpu.VMEM((2, _CHUNK, 2 * dh), jnp.float32),
            pltpu.VMEM((2, _CHUNK, dh), jnp.float32),
            pltpu.VMEM((2, _CHUNK, DEN_W), jnp.float32),
            pltpu.VMEM_SHARED((N_PAD, dh), jnp.float32),
            pltpu.VMEM_SHARED((N_PAD, DEN_W), jnp.float32),
            pltpu.SemaphoreType.DMA,
            pltpu.SemaphoreType.DMA,
            pltpu.SemaphoreType.DMA,
            pltpu.SemaphoreType.DMA,
        ],
    )


def _proj_body(x_ref, w_ref, b_ref, o_ref):
    o_ref[...] = (jnp.dot(x_ref[...], w_ref[...],
                          preferred_element_type=jnp.float32) + b_ref[...])


def _mid_body(na_ref, nb_ref, da_ref, db_ref, sel_ref, s_ref, w_ref, b_ref,
              o_ref):
    num = na_ref[...] + nb_ref[...]
    den = jnp.dot(da_ref[...] + db_ref[...], sel_ref[...],
                  preferred_element_type=jnp.float32)
    h = jnp.maximum(num / (den + 1e-16) + s_ref[...], 0.0)
    o_ref[...] = (jnp.dot(h, w_ref[...],
                          preferred_element_type=jnp.float32) + b_ref[...])


def _tail_body(na_ref, nb_ref, da_ref, db_ref, sel_ref, s_ref, batch_ref,
               w_ref, b_ref, o_ref, sums_ref, cnts_ref):
    i = pl.program_id(0)

    @pl.when(i == 0)
    def _():
        sums_ref[...] = jnp.zeros_like(sums_ref)
        cnts_ref[...] = jnp.zeros_like(cnts_ref)

    num = na_ref[...] + nb_ref[...]
    den = jnp.dot(da_ref[...] + db_ref[...], sel_ref[...],
                  preferred_element_type=jnp.float32)
    h = jnp.maximum(num / (den + 1e-16) + s_ref[...], 0.0)
    bv = batch_ref[0, 0, :].reshape(1, _ROW_BLK)
    gids = lax.broadcasted_iota(jnp.int32, (NUM_GRAPHS, _ROW_BLK), 0)
    onehot_t = (gids == bv).astype(jnp.float32)
    sums_ref[...] += jnp.dot(onehot_t, h, preferred_element_type=jnp.float32)
    cnts_ref[...] += jnp.dot(onehot_t, jnp.ones((_ROW_BLK, 1), jnp.float32),
                             preferred_element_type=jnp.float32)

    @pl.when(i == _NBLK - 1)
    def _():
        pooled = sums_ref[...] / jnp.maximum(cnts_ref[...], 1.0)
        o_ref[...] = (jnp.dot(pooled, w_ref[...],
                              preferred_element_type=jnp.float32) + b_ref[...])


def _proj_call(x_pad, wcat, bcat, d_in, d_out):
    return pl.pallas_call(
        _proj_body,
        grid=(_NBLK,),
        in_specs=[
            pl.BlockSpec((_ROW_BLK, d_in), lambda i: (i, 0)),
            pl.BlockSpec((d_in, d_out), lambda i: (0, 0)),
            pl.BlockSpec((1, d_out), lambda i: (0, 0)),
        ],
        out_specs=pl.BlockSpec((_ROW_BLK, d_out), lambda i: (i, 0)),
        out_shape=jax.ShapeDtypeStruct((N_PAD, d_out), jnp.float32),
    )(x_pad, wcat, bcat)


def _mid_call(na, nb, da, db, sel, s1, wcat, bcat, dh, d_out):
    node = pl.BlockSpec((_ROW_BLK, dh), lambda i: (i, 0))
    dnode = pl.BlockSpec((_ROW_BLK, DEN_W), lambda i: (i, 0))
    return pl.pallas_call(
        _mid_body,
        grid=(_NBLK,),
        in_specs=[
            node, node, dnode, dnode,
            pl.BlockSpec((DEN_W, dh), lambda i: (0, 0)),
            node,
            pl.BlockSpec((dh, d_out), lambda i: (0, 0)),
            pl.BlockSpec((1, d_out), lambda i: (0, 0)),
        ],
        out_specs=pl.BlockSpec((_ROW_BLK, d_out), lambda i: (i, 0)),
        out_shape=jax.ShapeDtypeStruct((N_PAD, d_out), jnp.float32),
    )(na, nb, da, db, sel, s1, wcat, bcat)


def _tail_call(na, nb, da, db, sel, s2, batch3, wfc_t, bfc, dh):
    node = pl.BlockSpec((_ROW_BLK, dh), lambda i: (i, 0))
    dnode = pl.BlockSpec((_ROW_BLK, DEN_W), lambda i: (i, 0))
    return pl.pallas_call(
        _tail_body,
        grid=(_NBLK,),
        in_specs=[
            node, node, dnode, dnode,
            pl.BlockSpec((DEN_W, dh), lambda i: (0, 0)),
            node,
            pl.BlockSpec((1, 1, _ROW_BLK), lambda i: (i, 0, 0)),
            pl.BlockSpec((dh, OUT_DIM), lambda i: (0, 0)),
            pl.BlockSpec((1, OUT_DIM), lambda i: (0, 0)),
        ],
        out_specs=pl.BlockSpec((NUM_GRAPHS, OUT_DIM), lambda i: (0, 0)),
        out_shape=jax.ShapeDtypeStruct((NUM_GRAPHS, OUT_DIM), jnp.float32),
        scratch_shapes=[
            pltpu.VMEM((NUM_GRAPHS, 16), jnp.float32),
            pltpu.VMEM((NUM_GRAPHS, 1), jnp.float32),
        ],
    )(na, nb, da, db, sel, s2, batch3, wfc_t, bfc)


_edge_pass_32 = _make_edge_pass(32)
_edge_pass_16 = _make_edge_pass(16)


def _den_selector(dh):
    sel = jnp.repeat(jnp.eye(HEADS, dtype=jnp.float32), dh // HEADS, axis=1)
    return jnp.concatenate(
        [sel, jnp.zeros((DEN_W - HEADS, dh), jnp.float32)], axis=0)


def kernel(x, edge_index, batch, Wq1, bq1, Wk1, bk1, Wv1, bv1, Ws1, bs1,
           Wq2, bq2, Wk2, bk2, Wv2, bv2, Ws2, bs2, Wfc, bfc):
    src = jnp.concatenate([edge_index[0].astype(jnp.int32),
                           jnp.zeros((E_PAD - N_EDGES,), jnp.int32)])
    dst = jnp.concatenate([edge_index[1].astype(jnp.int32),
                           jnp.full((E_PAD - N_EDGES,), N_NODES, jnp.int32)])
    x_pad = jnp.zeros((N_PAD, D_FEAT), jnp.float32).at[:N_NODES].set(x)

    w1 = jnp.concatenate([Wq1, Wk1, Wv1, Ws1], axis=0).T
    b1 = jnp.concatenate([bq1, bk1, bv1, bs1]).reshape(1, -1)
    out1 = _proj_call(x_pad, w1, b1, D_FEAT, 128)
    q1, kv1, s1 = out1[:, :32], out1[:, 32:96], out1[:, 96:128]

    na, da, nb, db = _edge_pass_32(q1, kv1, src, dst)

    w2 = jnp.concatenate([Wq2, Wk2, Wv2, Ws2], axis=0).T
    b2 = jnp.concatenate([bq2, bk2, bv2, bs2]).reshape(1, -1)
    out2 = _mid_call(na, nb, da, db, _den_selector(32), s1, w2, b2, 32, 64)
    q2, kv2, s2 = out2[:, :16], out2[:, 16:48], out2[:, 48:64]

    na2, da2, nb2, db2 = _edge_pass_16(q2, kv2, src, dst)

    batch3 = jnp.concatenate(
        [batch.astype(jnp.int32),
         jnp.full((N_PAD - N_NODES,), NUM_GRAPHS, jnp.int32)]
    ).reshape(_NBLK, 1, _ROW_BLK)
    return _tail_call(na2, nb2, da2, db2, _den_selector(16), s2, batch3,
                      Wfc.T, bfc.reshape(1, -1), 16)
